# baseline (speedup 1.0000x reference)
import functools

import jax
import jax.numpy as jnp
from jax import lax
from jax.experimental import pallas as pl
from jax.experimental.pallas import tpu as pltpu
from jax.experimental.pallas import tpu_sc as plsc


def kernel(tokens, emb_table, pos_table):
    B, T = tokens.shape
    V, D = emb_table.shape
    info = plsc.get_sparse_core_info()
    NC, NS = info.num_cores, info.num_subcores
    NW = NC * NS
    N = B * T
    b_per_w = N // NW
    K = 128
    nk = b_per_w // K

    tok2d = tokens.reshape(N // K, K)

    mesh = plsc.VectorSubcoreMesh(core_axis_name="c", subcore_axis_name="s")

    @functools.partial(
        pl.kernel,
        mesh=mesh,
        out_type=jax.ShapeDtypeStruct((N, D), jnp.float32),
        compiler_params=pltpu.CompilerParams(use_tc_tiling_on_sc=False),
        scratch_types=[
            pltpu.VMEM((nk, K), jnp.int32),
            pltpu.VMEM((b_per_w, D), jnp.float32),
            pltpu.VMEM((b_per_w, D), jnp.float32),
            pltpu.SemaphoreType.DMA,
        ],
    )
    def emb_kernel(tok_hbm, table_hbm, pos_hbm, out_hbm, idx_v, rows_v, pos_v, sem):
        wid = lax.axis_index("s") * NC + lax.axis_index("c")
        base = wid * b_per_w
        pltpu.sync_copy(tok_hbm.at[pl.ds(wid * nk, nk)], idx_v)
        cps = [
            pltpu.async_copy(table_hbm.at[idx_v.at[j]],
                             rows_v.at[pl.ds(j * K, K)], sem)
            for j in range(nk)
        ]
        t_base = lax.rem(base, T)
        pltpu.sync_copy(pos_hbm.at[pl.ds(t_base, b_per_w)], pos_v)
        for cp in cps:
            cp.wait()

        def add_row(r, carry):
            for j in range(D // 16):
                sl = pl.ds(j * 16, 16)
                rows_v[r, sl] = rows_v[r, sl] + pos_v[r, sl]
            return carry

        lax.fori_loop(0, b_per_w, add_row, 0)
        pltpu.sync_copy(rows_v, out_hbm.at[pl.ds(base, b_per_w)])

    out = emb_kernel(tok2d, emb_table, pos_table)
    return out.reshape(B, T, D)


# speedup vs baseline: 2.5592x; 2.5592x over previous
import functools

import jax
import jax.numpy as jnp
from jax import lax
from jax.experimental import pallas as pl
from jax.experimental.pallas import tpu as pltpu
from jax.experimental.pallas import tpu_sc as plsc


def kernel(tokens, emb_table, pos_table):
    B, T = tokens.shape
    V, D = emb_table.shape
    info = plsc.get_sparse_core_info()
    NC, NS = info.num_cores, info.num_subcores
    NW = NC * NS
    N = B * T
    b_per_w = N // NW
    L = 16

    table3 = emb_table.reshape(V // 8, 8, D)
    tok2d = tokens.reshape(N // 128, 128)

    mesh = plsc.VectorSubcoreMesh(core_axis_name="c", subcore_axis_name="s")

    @functools.partial(
        pl.kernel,
        mesh=mesh,
        out_type=jax.ShapeDtypeStruct((N, D), jnp.float32),
        scratch_types=[
            pltpu.VMEM((N // 128, 128), jnp.int32),
            pltpu.VMEM((b_per_w, D), jnp.float32),
            pltpu.VMEM((b_per_w, D), jnp.float32),
            pltpu.SemaphoreType.DMA,
            pltpu.SemaphoreType.DMA,
        ],
    )
    def emb_kernel(tok_hbm, table_hbm, pos_hbm, out_hbm,
                   tok_v, rows_v, pos_v, sem, psem):
        wid = lax.axis_index("s") * NC + lax.axis_index("c")
        base = wid * b_per_w
        row0 = base // 128
        pltpu.sync_copy(tok_hbm, tok_v)
        t_base = lax.rem(base, T)
        pos_cp = pltpu.async_copy(pos_hbm.at[pl.ds(t_base, b_per_w)],
                                  pos_v, psem)

        def fire_group(g, carry):
            row = row0 + lax.div(g, jnp.int32(8))
            col = lax.rem(g, jnp.int32(8)) * L
            tokvec = tok_v[row, pl.ds(col, L)]
            hivec = lax.shift_right_logical(tokvec, 3)
            mvec = lax.rem(tokvec, 8)
            for i in range(L):
                r = g * L + i
                pltpu.async_copy(table_hbm.at[hivec[i], mvec[i]],
                                 rows_v.at[r], sem)
            return carry

        lax.fori_loop(0, b_per_w // L, fire_group, 0)
        pltpu.make_async_copy(out_hbm.at[pl.ds(base, b_per_w)],
                              rows_v, sem).wait()
        pos_cp.wait()

        def add_row(r, carry):
            for j in range(D // L):
                sl = pl.ds(j * L, L)
                rows_v[r, sl] = rows_v[r, sl] + pos_v[r, sl]
            return carry

        lax.fori_loop(0, b_per_w, add_row, 0)
        pltpu.sync_copy(rows_v, out_hbm.at[pl.ds(base, b_per_w)])

    out = emb_kernel(tok2d, table3, pos_table)
    return out.reshape(B, T, D)
